# Initial kernel scaffold; baseline (speedup 1.0000x reference)
import jax
import jax.numpy as jnp
from jax.experimental import pallas as pl


def kernel(x, edge_index, W1, b1, W2, b2, W3, b3):
    raise NotImplementedError("write your pallas kernel here")


# speedup vs baseline: 19.0151x; 19.0151x over previous
import jax
import jax.numpy as jnp
from jax import lax
from jax.experimental import pallas as pl
from jax.experimental.pallas import tpu as pltpu
from jax.experimental.pallas import tpu_sc as plsc

N = 50000
NP = 51200
E = 800000
NSUB = 16
RPT = NP // NSUB
ZR = 256
R = 400
G = N // R
NPB = NP // R
F32 = jnp.float32
_SC_PARAMS = pltpu.CompilerParams(use_tc_tiling_on_sc=False)


def _zero_fill(buf, rows, d):
    z = jnp.zeros((16,), F32)

    def row(i, _):
        for j in range(d // 16):
            buf[i, pl.ds(16 * j, 16)] = z
        return 0

    lax.fori_loop(0, rows, row, 0)


def _zero_acc_slice(acc, zbuf, rlo):
    for k in range(RPT // ZR):
        pltpu.sync_copy(zbuf, acc.at[pl.ds(rlo + k * ZR, ZR)])
    rem = RPT % ZR
    if rem:
        pltpu.sync_copy(zbuf.at[pl.ds(0, rem)],
                        acc.at[pl.ds(rlo + (RPT // ZR) * ZR, rem)])


def _sel8(core, a, b):
    return pl.multiple_of(jnp.where(core == 0, a, b), 8)


def _sc_agg(n_g, n_out, D, B, rounds, ecount):
    mesh = plsc.VectorSubcoreMesh(core_axis_name="c", subcore_axis_name="s")
    out_type = jax.ShapeDtypeStruct((n_out * NP, D), F32)
    scratch = [
        pltpu.VMEM((B,), jnp.int32),
        pltpu.VMEM((B,), jnp.int32),
        pltpu.VMEM((B, D), F32),
        pltpu.VMEM((ZR, D), F32),
        pltpu.VMEM_SHARED((NP, D), F32),
        pltpu.SemaphoreType.DMA,
    ]
    per = ecount // NSUB

    def body(g_hbm, src_hbm, dst_hbm, out, srcb, dstb, msg, zbuf, acc, sem):
        core = lax.axis_index("c")
        sub = lax.axis_index("s")
        _zero_fill(zbuf, ZR, D)
        rlo = sub * RPT

        for (c0, o0, elo0), (c1, o1, elo1) in rounds:
            cbase = _sel8(core, c0 * N, c1 * N)
            obase = _sel8(core, o0 * NP, o1 * NP)
            elo = _sel8(core, elo0, elo1)
            _zero_acc_slice(acc, zbuf, rlo)
            plsc.subcore_barrier()
            e0 = elo + sub * per

            def blk(i, _):
                off = pl.multiple_of(e0 + i * B, 8)
                pltpu.sync_copy(src_hbm.at[pl.ds(off, B)], srcb)
                pltpu.sync_copy(dst_hbm.at[pl.ds(off, B)], dstb)
                pltpu.async_copy(
                    g_hbm.at[pl.ds(cbase, N)].at[srcb], msg, sem).wait()
                pltpu.sync_copy(msg, acc.at[dstb], add=True)
                return 0

            lax.fori_loop(0, per // B, blk, 0)
            plsc.subcore_barrier()
            pltpu.sync_copy(acc.at[pl.ds(rlo, RPT)],
                            out.at[pl.ds(obase + rlo, RPT)])

    return pl.kernel(body, out_type=out_type, mesh=mesh,
                     scratch_types=scratch, compiler_params=_SC_PARAMS)


def _sc_hist(B=1000):
    D = 16
    mesh = plsc.VectorSubcoreMesh(core_axis_name="c", subcore_axis_name="s")
    out_type = jax.ShapeDtypeStruct((2 * NP, D), F32)
    scratch = [
        pltpu.VMEM((B,), jnp.int32),
        pltpu.VMEM((B, D), F32),
        pltpu.VMEM((ZR, D), F32),
        pltpu.VMEM_SHARED((NP, D), F32),
    ]
    per = (E // 2) // NSUB

    def body(dst_hbm, out, dstb, ones, zbuf, acc):
        core = lax.axis_index("c")
        sub = lax.axis_index("s")
        _zero_fill(zbuf, ZR, D)
        o = jnp.ones((16,), F32)

        def orow(i, _):
            ones[i, pl.ds(0, 16)] = o
            return 0

        lax.fori_loop(0, B, orow, 0)

        rlo = sub * RPT
        _zero_acc_slice(acc, zbuf, rlo)
        plsc.subcore_barrier()
        e0 = core * (E // 2) + sub * per

        def blk(i, _):
            off = pl.multiple_of(e0 + i * B, 8)
            pltpu.sync_copy(dst_hbm.at[pl.ds(off, B)], dstb)
            pltpu.sync_copy(ones, acc.at[dstb], add=True)
            return 0

        lax.fori_loop(0, per // B, blk, 0)
        plsc.subcore_barrier()
        obase = pl.multiple_of(core * NP, 8)
        pltpu.sync_copy(acc.at[pl.ds(rlo, RPT)],
                        out.at[pl.ds(obase + rlo, RPT)])

    return pl.kernel(body, out_type=out_type, mesh=mesh,
                     scratch_types=scratch, compiler_params=_SC_PARAMS)


def _rowspec(d, base_blocks=0):
    if base_blocks:
        return pl.BlockSpec((R, d), lambda i, b=base_blocks: (b + i, 0))
    return pl.BlockSpec((R, d), lambda i: (i, 0))


def _fullspec(r, c):
    return pl.BlockSpec((r, c), lambda i: (0, 0))


def _tc1(hist, x):
    def body(h0_, h1_, x_, dinv, g0):
        deg = h0_[:, 0:1] + h1_[:, 0:1] + 1.0
        di = lax.rsqrt(deg)
        dinv[...] = di
        g = x_[...] * di
        g0[...] = jnp.stack(
            [g[:, 0:32],
             jnp.concatenate([g[:, 32:58], jnp.zeros((R, 6), F32)], axis=1)])

    return pl.pallas_call(
        body,
        grid=(G,),
        in_specs=[_rowspec(16), _rowspec(16, NPB), _rowspec(58)],
        out_specs=[_rowspec(1), pl.BlockSpec((2, R, 32), lambda i: (0, i, 0))],
        out_shape=[jax.ShapeDtypeStruct((N, 1), F32),
                   jax.ShapeDtypeStruct((2, N, 32), F32)],
    )(hist, hist, x)


def _tc2(s0, g0, dinv, W1p, b1p, W2p):
    def body(s0a, s0b, g0a, g0b, di_, W1_, b1_, W2_, g1):
        di = di_[...]
        a = jnp.concatenate(
            [s0a[...] + g0a[...], s0b[...] + g0b[...]], axis=1) * di
        h = jnp.maximum(
            jnp.dot(a, W1_[...], preferred_element_type=F32) + b1_[...], 0.0)
        p = jnp.dot(h, W2_[...], preferred_element_type=F32) * di
        g1[...] = jnp.stack([p[:, 32 * c:32 * c + 32] for c in range(4)])

    return pl.pallas_call(
        body,
        grid=(G,),
        in_specs=[_rowspec(32), _rowspec(32, NPB),
                  _rowspec(32), _rowspec(32, G),
                  _rowspec(1), _fullspec(64, 384), _fullspec(1, 384),
                  _fullspec(384, 128)],
        out_specs=[pl.BlockSpec((4, R, 32), lambda i: (0, i, 0))],
        out_shape=[jax.ShapeDtypeStruct((4, N, 32), F32)],
    )(s0, s0, g0, g0, dinv, W1p, b1p, W2p)


def _tc3(s1, g1, dinv, b2p, W3p):
    def body(sa, sb, sc, sd, ga, gb, gc, gd, di_, b2_, W3_, out):
        di = di_[...]
        a = jnp.concatenate(
            [sa[...] + ga[...], sb[...] + gb[...],
             sc[...] + gc[...], sd[...] + gd[...]], axis=1) * di
        h = jnp.maximum(a + b2_[...], 0.0)
        out[...] = jnp.dot(h, W3_[...], preferred_element_type=F32) * di

    return pl.pallas_call(
        body,
        grid=(G,),
        in_specs=[_rowspec(32), _rowspec(32, NPB), _rowspec(32, 2 * NPB),
                  _rowspec(32, 3 * NPB),
                  _rowspec(32), _rowspec(32, G), _rowspec(32, 2 * G),
                  _rowspec(32, 3 * G),
                  _rowspec(1), _fullspec(1, 128), _fullspec(128, 16)],
        out_specs=[_rowspec(16)],
        out_shape=[jax.ShapeDtypeStruct((N, 16), F32)],
    )(s1, s1, s1, s1, g1, g1, g1, g1, dinv, b2p, W3p)


def _tc4(s2, g2, dinv, b3p):
    def body(sa, sb, g2_, di_, b3_, out):
        t = (sa[...] + sb[...] + g2_[...]) * di_[...]
        out[...] = t[:, 0:1] + b3_[...]

    return pl.pallas_call(
        body,
        grid=(G,),
        in_specs=[_rowspec(16), _rowspec(16, NPB), _rowspec(16), _rowspec(1),
                  _fullspec(1, 1)],
        out_specs=[_rowspec(1)],
        out_shape=[jax.ShapeDtypeStruct((N, 1), F32)],
    )(s2, s2, g2, dinv, b3p)


_hist = _sc_hist()
_agg1 = _sc_agg(n_g=2, n_out=2, D=32, B=400,
                rounds=[((0, 0, 0), (1, 1, 0))], ecount=E)
_agg2 = _sc_agg(n_g=4, n_out=4, D=32, B=400,
                rounds=[((0, 0, 0), (1, 1, 0)), ((2, 2, 0), (3, 3, 0))],
                ecount=E)
_agg3 = _sc_agg(n_g=1, n_out=2, D=16, B=1000,
                rounds=[((0, 0, 0), (0, 1, E // 2))], ecount=E // 2)


def kernel(x, edge_index, W1, b1, W2, b2, W3, b3):
    src = edge_index[0]
    dst = edge_index[1]
    W1p = jnp.zeros((64, 384), F32).at[:58, :300].set(W1)
    b1p = jnp.zeros((1, 384), F32).at[0, :300].set(b1)
    W2p = jnp.zeros((384, 128), F32).at[:300, :100].set(W2)
    b2p = jnp.zeros((1, 128), F32).at[0, :100].set(b2)
    W3p = jnp.zeros((128, 16), F32).at[:100, 0:1].set(W3)
    b3p = b3.reshape(1, 1)

    hist = _hist(dst)
    dinv, g0 = _tc1(hist, x)
    g0f = g0.reshape(2 * N, 32)
    s0 = _agg1(g0f, src, dst)
    (g1,) = _tc2(s0, g0f, dinv, W1p, b1p, W2p)
    g1f = g1.reshape(4 * N, 32)
    s1 = _agg2(g1f, src, dst)
    (g2,) = _tc3(s1, g1f, dinv, b2p, W3p)
    s2 = _agg3(g2, src, dst)
    (out,) = _tc4(s2, g2, dinv, b3p)
    return out


# speedup vs baseline: 26.7788x; 1.4083x over previous
import jax
import jax.numpy as jnp
from jax import lax
from jax.experimental import pallas as pl
from jax.experimental.pallas import tpu as pltpu
from jax.experimental.pallas import tpu_sc as plsc

N = 50000
NP = 51200
E = 800000
NSUB = 16
RPT = NP // NSUB
NA = 50048
RPTA = NA // NSUB
ZR = 64
R = 400
G = N // R
NPB = NP // R
F32 = jnp.float32
_SC_PARAMS = pltpu.CompilerParams(use_tc_tiling_on_sc=False)


def _zero_fill(buf, rows, d):
    z = jnp.zeros((16,), F32)

    def row(i, _):
        for j in range(d // 16):
            buf[i, pl.ds(16 * j, 16)] = z
        return 0

    lax.fori_loop(0, rows, row, 0)


def _zero_acc_slice(acc, zbuf, rlo, rows):
    for k in range(rows // ZR):
        pltpu.sync_copy(zbuf, acc.at[pl.ds(rlo + k * ZR, ZR)])
    rem = rows % ZR
    if rem:
        pltpu.sync_copy(zbuf.at[pl.ds(0, rem)],
                        acc.at[pl.ds(rlo + (rows // ZR) * ZR, rem)])


def _sel8(core, a, b):
    return pl.multiple_of(jnp.where(core == 0, a, b), 8)


def _sc_agg(n_g, n_out, D, B, rounds, ecount):
    mesh = plsc.VectorSubcoreMesh(core_axis_name="c", subcore_axis_name="s")
    out_type = jax.ShapeDtypeStruct((n_out * NP, D), F32)
    scratch = [
        pltpu.VMEM((B,), jnp.int32),
        pltpu.VMEM((B,), jnp.int32),
        pltpu.VMEM((B,), jnp.int32),
        pltpu.VMEM((B,), jnp.int32),
        pltpu.VMEM((B, D), F32),
        pltpu.VMEM((B, D), F32),
        pltpu.VMEM((ZR, D), F32),
        pltpu.VMEM_SHARED((NA, D), F32),
    ] + [pltpu.SemaphoreType.DMA] * 6
    per = ecount // NSUB
    nblk = per // B
    assert nblk * B == per and B % 8 == 0
    pairs, odd = nblk // 2, nblk % 2

    def body(g_hbm, src_hbm, dst_hbm, out,
             srcb0, dstb0, srcb1, dstb1, msg0, msg1, zbuf, acc,
             si0, si1, sg0, sg1, ss0, ss1):
        core = lax.axis_index("c")
        sub = lax.axis_index("s")
        _zero_fill(zbuf, ZR, D)
        rlo = sub * RPTA

        def idx_start(a, sb, db, sem):
            off = pl.multiple_of(a, 8)
            pltpu.async_copy(src_hbm.at[pl.ds(off, B)], sb, sem)
            pltpu.async_copy(dst_hbm.at[pl.ds(off, B)], db, sem)

        def idx_wait(sb, db, sem):
            pltpu.make_async_copy(src_hbm.at[pl.ds(0, B)], sb, sem).wait()
            pltpu.make_async_copy(dst_hbm.at[pl.ds(0, B)], db, sem).wait()

        for (c0, o0, elo0), (c1, o1, elo1) in rounds:
            cbase = _sel8(core, c0 * N, c1 * N)
            obase = _sel8(core, o0 * NP, o1 * NP)
            elo = _sel8(core, elo0, elo1)
            _zero_acc_slice(acc, zbuf, rlo, RPTA)
            plsc.subcore_barrier()
            e0 = elo + sub * per

            def off(a):
                return e0 + jnp.minimum(a, nblk - 1) * B

            def gat(sb, m, sem):
                pltpu.async_copy(g_hbm.at[pl.ds(cbase, N)].at[sb], m, sem)

            def gat_wait(sb, m, sem):
                pltpu.make_async_copy(
                    g_hbm.at[pl.ds(cbase, N)].at[sb], m, sem).wait()

            def sca(m, db, sem):
                pltpu.async_copy(m, acc.at[db], sem, add=True)

            def sca_wait(m, db, sem):
                pltpu.make_async_copy(m, acc.at[db], sem).wait()

            idx_start(off(0), srcb0, dstb0, si0)
            idx_wait(srcb0, dstb0, si0)
            idx_start(off(1), srcb1, dstb1, si1)
            gat(srcb0, msg0, sg0)

            def pair(k, _):
                a = 2 * k
                gat_wait(srcb0, msg0, sg0)
                sca(msg0, dstb0, ss0)
                idx_wait(srcb1, dstb1, si1)
                gat(srcb1, msg1, sg1)
                sca_wait(msg0, dstb0, ss0)
                idx_start(off(a + 2), srcb0, dstb0, si0)
                gat_wait(srcb1, msg1, sg1)
                sca(msg1, dstb1, ss1)
                idx_wait(srcb0, dstb0, si0)
                gat(srcb0, msg0, sg0)
                sca_wait(msg1, dstb1, ss1)
                idx_start(off(a + 3), srcb1, dstb1, si1)
                return 0

            lax.fori_loop(0, pairs, pair, 0)
            gat_wait(srcb0, msg0, sg0)
            if odd:
                sca(msg0, dstb0, ss0)
                sca_wait(msg0, dstb0, ss0)
            idx_wait(srcb1, dstb1, si1)
            plsc.subcore_barrier()
            pltpu.sync_copy(acc.at[pl.ds(rlo, RPTA)],
                            out.at[pl.ds(obase + rlo, RPTA)])

    return pl.kernel(body, out_type=out_type, mesh=mesh,
                     scratch_types=scratch, compiler_params=_SC_PARAMS)


def _sc_hist(B=1000):
    D = 16
    mesh = plsc.VectorSubcoreMesh(core_axis_name="c", subcore_axis_name="s")
    out_type = jax.ShapeDtypeStruct((2 * NP, D), F32)
    scratch = [
        pltpu.VMEM((B,), jnp.int32),
        pltpu.VMEM((B, D), F32),
        pltpu.VMEM((ZR, D), F32),
        pltpu.VMEM_SHARED((NP, D), F32),
    ]
    per = (E // 2) // NSUB

    def body(dst_hbm, out, dstb, ones, zbuf, acc):
        core = lax.axis_index("c")
        sub = lax.axis_index("s")
        _zero_fill(zbuf, ZR, D)
        o = jnp.ones((16,), F32)

        def orow(i, _):
            ones[i, pl.ds(0, 16)] = o
            return 0

        lax.fori_loop(0, B, orow, 0)

        rlo = sub * RPT
        _zero_acc_slice(acc, zbuf, rlo, RPT)
        plsc.subcore_barrier()
        e0 = core * (E // 2) + sub * per

        def blk(i, _):
            off = pl.multiple_of(e0 + i * B, 8)
            pltpu.sync_copy(dst_hbm.at[pl.ds(off, B)], dstb)
            pltpu.sync_copy(ones, acc.at[dstb], add=True)
            return 0

        lax.fori_loop(0, per // B, blk, 0)
        plsc.subcore_barrier()
        obase = pl.multiple_of(core * NP, 8)
        pltpu.sync_copy(acc.at[pl.ds(rlo, RPT)],
                        out.at[pl.ds(obase + rlo, RPT)])

    return pl.kernel(body, out_type=out_type, mesh=mesh,
                     scratch_types=scratch, compiler_params=_SC_PARAMS)


def _rowspec(d, base_blocks=0):
    if base_blocks:
        return pl.BlockSpec((R, d), lambda i, b=base_blocks: (b + i, 0))
    return pl.BlockSpec((R, d), lambda i: (i, 0))


def _fullspec(r, c):
    return pl.BlockSpec((r, c), lambda i: (0, 0))


def _tc1(hist, x):
    def body(h0_, h1_, x_, dinv, g0):
        deg = h0_[:, 0:1] + h1_[:, 0:1] + 1.0
        di = lax.rsqrt(deg)
        dinv[...] = di
        g = x_[...] * di
        g0[...] = jnp.stack(
            [g[:, 0:32],
             jnp.concatenate([g[:, 32:58], jnp.zeros((R, 6), F32)], axis=1)])

    return pl.pallas_call(
        body,
        grid=(G,),
        in_specs=[_rowspec(16), _rowspec(16, NPB), _rowspec(58)],
        out_specs=[_rowspec(1), pl.BlockSpec((2, R, 32), lambda i: (0, i, 0))],
        out_shape=[jax.ShapeDtypeStruct((N, 1), F32),
                   jax.ShapeDtypeStruct((2, N, 32), F32)],
    )(hist, hist, x)


def _tc2(s0, g0, dinv, W1p, b1p, W2p):
    def body(s0a, s0b, g0a, g0b, di_, W1_, b1_, W2_, g1):
        di = di_[...]
        a = jnp.concatenate(
            [s0a[...] + g0a[...], s0b[...] + g0b[...]], axis=1) * di
        h = jnp.maximum(
            jnp.dot(a, W1_[...], preferred_element_type=F32) + b1_[...], 0.0)
        p = jnp.dot(h, W2_[...], preferred_element_type=F32) * di
        g1[...] = jnp.stack([p[:, 32 * c:32 * c + 32] for c in range(4)])

    return pl.pallas_call(
        body,
        grid=(G,),
        in_specs=[_rowspec(32), _rowspec(32, NPB),
                  _rowspec(32), _rowspec(32, G),
                  _rowspec(1), _fullspec(64, 384), _fullspec(1, 384),
                  _fullspec(384, 128)],
        out_specs=[pl.BlockSpec((4, R, 32), lambda i: (0, i, 0))],
        out_shape=[jax.ShapeDtypeStruct((4, N, 32), F32)],
    )(s0, s0, g0, g0, dinv, W1p, b1p, W2p)


def _tc3(s1, g1, dinv, b2p, W3p):
    def body(sa, sb, sc, sd, ga, gb, gc, gd, di_, b2_, W3_, out):
        di = di_[...]
        a = jnp.concatenate(
            [sa[...] + ga[...], sb[...] + gb[...],
             sc[...] + gc[...], sd[...] + gd[...]], axis=1) * di
        h = jnp.maximum(a + b2_[...], 0.0)
        out[...] = jnp.dot(h, W3_[...], preferred_element_type=F32) * di

    return pl.pallas_call(
        body,
        grid=(G,),
        in_specs=[_rowspec(32), _rowspec(32, NPB), _rowspec(32, 2 * NPB),
                  _rowspec(32, 3 * NPB),
                  _rowspec(32), _rowspec(32, G), _rowspec(32, 2 * G),
                  _rowspec(32, 3 * G),
                  _rowspec(1), _fullspec(1, 128), _fullspec(128, 16)],
        out_specs=[_rowspec(16)],
        out_shape=[jax.ShapeDtypeStruct((N, 16), F32)],
    )(s1, s1, s1, s1, g1, g1, g1, g1, dinv, b2p, W3p)


def _tc4(s2, g2, dinv, b3p):
    def body(sa, sb, g2_, di_, b3_, out):
        t = (sa[...] + sb[...] + g2_[...]) * di_[...]
        out[...] = t[:, 0:1] + b3_[...]

    return pl.pallas_call(
        body,
        grid=(G,),
        in_specs=[_rowspec(16), _rowspec(16, NPB), _rowspec(16), _rowspec(1),
                  _fullspec(1, 1)],
        out_specs=[_rowspec(1)],
        out_shape=[jax.ShapeDtypeStruct((N, 1), F32)],
    )(s2, s2, g2, dinv, b3p)


_hist = _sc_hist()
_agg1 = _sc_agg(n_g=2, n_out=2, D=32, B=400,
                rounds=[((0, 0, 0), (1, 1, 0))], ecount=E)
_agg2 = _sc_agg(n_g=4, n_out=4, D=32, B=400,
                rounds=[((0, 0, 0), (1, 1, 0)), ((2, 2, 0), (3, 3, 0))],
                ecount=E)
_agg3 = _sc_agg(n_g=1, n_out=2, D=16, B=1000,
                rounds=[((0, 0, 0), (0, 1, E // 2))], ecount=E // 2)


def kernel(x, edge_index, W1, b1, W2, b2, W3, b3):
    src = edge_index[0]
    dst = edge_index[1]
    W1p = jnp.zeros((64, 384), F32).at[:58, :300].set(W1)
    b1p = jnp.zeros((1, 384), F32).at[0, :300].set(b1)
    W2p = jnp.zeros((384, 128), F32).at[:300, :100].set(W2)
    b2p = jnp.zeros((1, 128), F32).at[0, :100].set(b2)
    W3p = jnp.zeros((128, 16), F32).at[:100, 0:1].set(W3)
    b3p = b3.reshape(1, 1)

    hist = _hist(dst)
    dinv, g0 = _tc1(hist, x)
    g0f = g0.reshape(2 * N, 32)
    s0 = _agg1(g0f, src, dst)
    (g1,) = _tc2(s0, g0f, dinv, W1p, b1p, W2p)
    g1f = g1.reshape(4 * N, 32)
    s1 = _agg2(g1f, src, dst)
    (g2,) = _tc3(s1, g1f, dinv, b2p, W3p)
    s2 = _agg3(g2, src, dst)
    (out,) = _tc4(s2, g2, dinv, b3p)
    return out


# speedup vs baseline: 30.6578x; 1.1449x over previous
import jax
import jax.numpy as jnp
from jax import lax
from jax.experimental import pallas as pl
from jax.experimental.pallas import tpu as pltpu
from jax.experimental.pallas import tpu_sc as plsc

N = 50000
NP = 64000
E = 800000
NSUB = 16
RPT = 51200 // NSUB
NA = 50048
RPTA = NA // NSUB
ZR = 64
R = 2000
G = N // R
NPB = NP // R
F32 = jnp.float32
_SC_PARAMS = pltpu.CompilerParams(use_tc_tiling_on_sc=False)


def _zero_fill(buf, rows, d):
    z = jnp.zeros((16,), F32)

    def row(i, _):
        for j in range(d // 16):
            buf[i, pl.ds(16 * j, 16)] = z
        return 0

    lax.fori_loop(0, rows, row, 0)


def _zero_acc_slice(acc, zbuf, rlo, rows):
    for k in range(rows // ZR):
        pltpu.sync_copy(zbuf, acc.at[pl.ds(rlo + k * ZR, ZR)])
    rem = rows % ZR
    if rem:
        pltpu.sync_copy(zbuf.at[pl.ds(0, rem)],
                        acc.at[pl.ds(rlo + (rows // ZR) * ZR, rem)])


def _sel8(core, a, b):
    return pl.multiple_of(jnp.where(core == 0, a, b), 8)


def _sc_agg(n_g, n_out, D, B, rounds, ecount):
    mesh = plsc.VectorSubcoreMesh(core_axis_name="c", subcore_axis_name="s")
    out_type = jax.ShapeDtypeStruct((n_out * NP, D), F32)
    scratch = [
        pltpu.VMEM((B,), jnp.int32),
        pltpu.VMEM((B,), jnp.int32),
        pltpu.VMEM((B,), jnp.int32),
        pltpu.VMEM((B,), jnp.int32),
        pltpu.VMEM((B, D), F32),
        pltpu.VMEM((B, D), F32),
        pltpu.VMEM((ZR, D), F32),
        pltpu.VMEM_SHARED((NA, D), F32),
    ] + [pltpu.SemaphoreType.DMA] * 6
    per = ecount // NSUB
    nblk = per // B
    assert nblk * B == per and B % 8 == 0
    pairs, odd = nblk // 2, nblk % 2

    def body(g_hbm, src_hbm, dst_hbm, out,
             srcb0, dstb0, srcb1, dstb1, msg0, msg1, zbuf, acc,
             si0, si1, sg0, sg1, ss0, ss1):
        core = lax.axis_index("c")
        sub = lax.axis_index("s")
        _zero_fill(zbuf, ZR, D)
        rlo = sub * RPTA

        def idx_start(a, sb, db, sem):
            off = pl.multiple_of(a, 8)
            pltpu.async_copy(src_hbm.at[pl.ds(off, B)], sb, sem)
            pltpu.async_copy(dst_hbm.at[pl.ds(off, B)], db, sem)

        def idx_wait(sb, db, sem):
            pltpu.make_async_copy(src_hbm.at[pl.ds(0, B)], sb, sem).wait()
            pltpu.make_async_copy(dst_hbm.at[pl.ds(0, B)], db, sem).wait()

        for (c0, o0, elo0), (c1, o1, elo1) in rounds:
            cidx = jnp.where(core == 0, c0, c1)
            obase = _sel8(core, o0 * NP, o1 * NP)
            elo = _sel8(core, elo0, elo1)
            _zero_acc_slice(acc, zbuf, rlo, RPTA)
            plsc.subcore_barrier()
            e0 = elo + sub * per

            def off(a):
                return e0 + jnp.minimum(a, nblk - 1) * B

            def gref(sb):
                if n_g == 1:
                    return g_hbm.at[sb]
                return g_hbm.at[cidx].at[sb]

            def gat(sb, m, sem):
                pltpu.async_copy(gref(sb), m, sem)

            def gat_wait(sb, m, sem):
                pltpu.make_async_copy(gref(sb), m, sem).wait()

            def sca(m, db, sem):
                pltpu.async_copy(m, acc.at[db], sem, add=True)

            def sca_wait(m, db, sem):
                pltpu.make_async_copy(m, acc.at[db], sem).wait()

            idx_start(off(0), srcb0, dstb0, si0)
            idx_wait(srcb0, dstb0, si0)
            idx_start(off(1), srcb1, dstb1, si1)
            gat(srcb0, msg0, sg0)

            def pair(k, _):
                a = 2 * k
                gat_wait(srcb0, msg0, sg0)
                sca(msg0, dstb0, ss0)
                idx_wait(srcb1, dstb1, si1)
                gat(srcb1, msg1, sg1)
                sca_wait(msg0, dstb0, ss0)
                idx_start(off(a + 2), srcb0, dstb0, si0)
                gat_wait(srcb1, msg1, sg1)
                sca(msg1, dstb1, ss1)
                idx_wait(srcb0, dstb0, si0)
                gat(srcb0, msg0, sg0)
                sca_wait(msg1, dstb1, ss1)
                idx_start(off(a + 3), srcb1, dstb1, si1)
                return 0

            lax.fori_loop(0, pairs, pair, 0)
            gat_wait(srcb0, msg0, sg0)
            if odd:
                sca(msg0, dstb0, ss0)
                sca_wait(msg0, dstb0, ss0)
            idx_wait(srcb1, dstb1, si1)
            plsc.subcore_barrier()
            pltpu.sync_copy(acc.at[pl.ds(rlo, RPTA)],
                            out.at[pl.ds(obase + rlo, RPTA)])

    return pl.kernel(body, out_type=out_type, mesh=mesh,
                     scratch_types=scratch, compiler_params=_SC_PARAMS)


def _sc_hist(B=1000):
    D = 16
    mesh = plsc.VectorSubcoreMesh(core_axis_name="c", subcore_axis_name="s")
    out_type = jax.ShapeDtypeStruct((2 * NP, D), F32)
    scratch = [
        pltpu.VMEM((B,), jnp.int32),
        pltpu.VMEM((B, D), F32),
        pltpu.VMEM((ZR, D), F32),
        pltpu.VMEM_SHARED((RPT * NSUB, D), F32),
    ]
    per = (E // 2) // NSUB

    def body(dst_hbm, out, dstb, ones, zbuf, acc):
        core = lax.axis_index("c")
        sub = lax.axis_index("s")
        _zero_fill(zbuf, ZR, D)
        o = jnp.ones((16,), F32)

        def orow(i, _):
            ones[i, pl.ds(0, 16)] = o
            return 0

        lax.fori_loop(0, B, orow, 0)

        rlo = sub * RPT
        _zero_acc_slice(acc, zbuf, rlo, RPT)
        plsc.subcore_barrier()
        e0 = core * (E // 2) + sub * per

        def blk(i, _):
            off = pl.multiple_of(e0 + i * B, 8)
            pltpu.sync_copy(dst_hbm.at[pl.ds(off, B)], dstb)
            pltpu.sync_copy(ones, acc.at[dstb], add=True)
            return 0

        lax.fori_loop(0, per // B, blk, 0)
        plsc.subcore_barrier()
        obase = pl.multiple_of(core * NP, 8)
        pltpu.sync_copy(acc.at[pl.ds(rlo, RPT)],
                        out.at[pl.ds(obase + rlo, RPT)])

    return pl.kernel(body, out_type=out_type, mesh=mesh,
                     scratch_types=scratch, compiler_params=_SC_PARAMS)


def _rowspec(d, base_blocks=0):
    if base_blocks:
        return pl.BlockSpec((R, d), lambda i, b=base_blocks: (b + i, 0))
    return pl.BlockSpec((R, d), lambda i: (i, 0))


def _fullspec(r, c):
    return pl.BlockSpec((r, c), lambda i: (0, 0))


def _tc1(hist, x):
    def body(h0_, h1_, x_, dinv, g0):
        deg = h0_[:, 0:1] + h1_[:, 0:1] + 1.0
        di = lax.rsqrt(deg)
        dinv[...] = di
        g = x_[...] * di
        g0[...] = jnp.stack(
            [g[:, 0:32],
             jnp.concatenate([g[:, 32:58], jnp.zeros((R, 6), F32)], axis=1)])

    return pl.pallas_call(
        body,
        grid=(G,),
        in_specs=[_rowspec(16), _rowspec(16, NPB), _rowspec(58)],
        out_specs=[_rowspec(1), pl.BlockSpec((2, R, 32), lambda i: (0, i, 0))],
        out_shape=[jax.ShapeDtypeStruct((N, 1), F32),
                   jax.ShapeDtypeStruct((2, N, 32), F32)],
    )(hist, hist, x)


def _tc2(s0, g0, dinv, W1p, b1p, W2p):
    def body(s0a, s0b, g0a, g0b, di_, W1_, b1_, W2_, g1):
        di = di_[...]
        a = jnp.concatenate(
            [s0a[...] + g0a[0], s0b[...] + g0b[0]], axis=1) * di
        h = jnp.maximum(
            jnp.dot(a, W1_[...], preferred_element_type=F32) + b1_[...], 0.0)
        p = jnp.dot(h, W2_[...], preferred_element_type=F32) * di
        g1[...] = jnp.stack([p[:, 32 * c:32 * c + 32] for c in range(4)])

    return pl.pallas_call(
        body,
        grid=(G,),
        in_specs=[_rowspec(32), _rowspec(32, NPB),
                  pl.BlockSpec((1, R, 32), lambda i: (0, i, 0)),
                  pl.BlockSpec((1, R, 32), lambda i: (1, i, 0)),
                  _rowspec(1), _fullspec(64, 384), _fullspec(1, 384),
                  _fullspec(384, 128)],
        out_specs=[pl.BlockSpec((4, R, 32), lambda i: (0, i, 0))],
        out_shape=[jax.ShapeDtypeStruct((4, N, 32), F32)],
    )(s0, s0, g0, g0, dinv, W1p, b1p, W2p)


def _tc3(s1, g1, dinv, b2p, W3p):
    def body(sa, sb, sc, sd, ga, gb, gc, gd, di_, b2_, W3_, out):
        di = di_[...]
        a = jnp.concatenate(
            [sa[...] + ga[0], sb[...] + gb[0],
             sc[...] + gc[0], sd[...] + gd[0]], axis=1) * di
        h = jnp.maximum(a + b2_[...], 0.0)
        out[...] = jnp.dot(h, W3_[...], preferred_element_type=F32) * di

    return pl.pallas_call(
        body,
        grid=(G,),
        in_specs=[_rowspec(32), _rowspec(32, NPB), _rowspec(32, 2 * NPB),
                  _rowspec(32, 3 * NPB)] +
                 [pl.BlockSpec((1, R, 32), lambda i, c=c: (c, i, 0))
                  for c in range(4)] +
                 [_rowspec(1), _fullspec(1, 128), _fullspec(128, 16)],
        out_specs=[_rowspec(16)],
        out_shape=[jax.ShapeDtypeStruct((N, 16), F32)],
    )(s1, s1, s1, s1, g1, g1, g1, g1, dinv, b2p, W3p)


def _tc4(s2, g2, dinv, b3p):
    def body(sa, sb, g2_, di_, b3_, out):
        t = (sa[...] + sb[...] + g2_[...]) * di_[...]
        out[...] = t[:, 0:1] + b3_[...]

    return pl.pallas_call(
        body,
        grid=(G,),
        in_specs=[_rowspec(16), _rowspec(16, NPB), _rowspec(16), _rowspec(1),
                  _fullspec(1, 1)],
        out_specs=[_rowspec(1)],
        out_shape=[jax.ShapeDtypeStruct((N, 1), F32)],
    )(s2, s2, g2, dinv, b3p)


_hist = _sc_hist()
_agg1 = _sc_agg(n_g=2, n_out=2, D=32, B=400,
                rounds=[((0, 0, 0), (1, 1, 0))], ecount=E)
_agg2 = _sc_agg(n_g=4, n_out=4, D=32, B=400,
                rounds=[((0, 0, 0), (1, 1, 0)), ((2, 2, 0), (3, 3, 0))],
                ecount=E)
_agg3 = _sc_agg(n_g=1, n_out=2, D=16, B=1000,
                rounds=[((0, 0, 0), (0, 1, E // 2))], ecount=E // 2)


def kernel(x, edge_index, W1, b1, W2, b2, W3, b3):
    src = edge_index[0]
    dst = edge_index[1]
    W1p = jnp.zeros((64, 384), F32).at[:58, :300].set(W1)
    b1p = jnp.zeros((1, 384), F32).at[0, :300].set(b1)
    W2p = jnp.zeros((384, 128), F32).at[:300, :100].set(W2)
    b2p = jnp.zeros((1, 128), F32).at[0, :100].set(b2)
    W3p = jnp.zeros((128, 16), F32).at[:100, 0:1].set(W3)
    b3p = b3.reshape(1, 1)

    hist = _hist(dst)
    dinv, g0 = _tc1(hist, x)
    s0 = _agg1(g0, src, dst)
    (g1,) = _tc2(s0, g0, dinv, W1p, b1p, W2p)
    s1 = _agg2(g1, src, dst)
    (g2,) = _tc3(s1, g1, dinv, b2p, W3p)
    s2 = _agg3(g2, src, dst)
    (out,) = _tc4(s2, g2, dinv, b3p)
    return out


# speedup vs baseline: 32.6054x; 1.0635x over previous
import jax
import jax.numpy as jnp
from jax import lax
from jax.experimental import pallas as pl
from jax.experimental.pallas import tpu as pltpu
from jax.experimental.pallas import tpu_sc as plsc

N = 50000
NP = 52000
E = 800000
NSUB = 16
RPT = 51200 // NSUB
NA = 50048
RPTA = NA // NSUB
ZR = 64
R = 2000
G = N // R
NPB = NP // R
F32 = jnp.float32
_SC_PARAMS = pltpu.CompilerParams(use_tc_tiling_on_sc=False)


def _zero_fill(buf, rows, d):
    z = jnp.zeros((16,), F32)

    def row(i, _):
        for j in range(d // 16):
            buf[i, pl.ds(16 * j, 16)] = z
        return 0

    lax.fori_loop(0, rows, row, 0)


def _zero_acc_slice(acc, zbuf, rlo, rows):
    for k in range(rows // ZR):
        pltpu.sync_copy(zbuf, acc.at[pl.ds(rlo + k * ZR, ZR)])
    rem = rows % ZR
    if rem:
        pltpu.sync_copy(zbuf.at[pl.ds(0, rem)],
                        acc.at[pl.ds(rlo + (rows // ZR) * ZR, rem)])


def _sel8(core, a, b):
    return pl.multiple_of(jnp.where(core == 0, a, b), 8)


def _sc_agg(n_g, n_out, D, B, rounds, ecount):
    mesh = plsc.VectorSubcoreMesh(core_axis_name="c", subcore_axis_name="s")
    out_type = jax.ShapeDtypeStruct((n_out * NP, D), F32)
    scratch = [
        pltpu.VMEM((B,), jnp.int32),
        pltpu.VMEM((B,), jnp.int32),
        pltpu.VMEM((B,), jnp.int32),
        pltpu.VMEM((B,), jnp.int32),
        pltpu.VMEM((B, D), F32),
        pltpu.VMEM((B, D), F32),
        pltpu.VMEM_SHARED((NA, D), F32),
    ] + [pltpu.SemaphoreType.DMA] * 6
    per = ecount // NSUB
    nblk = per // B
    assert nblk * B == per and B % 8 == 0
    pairs, odd = nblk // 2, nblk % 2

    def body(g_hbm, src_hbm, dst_hbm, out,
             srcb0, dstb0, srcb1, dstb1, msg0, msg1, acc,
             si0, si1, sg0, sg1, ss0, ss1):
        core = lax.axis_index("c")
        sub = lax.axis_index("s")
        rlo = sub * RPTA

        def idx_start(a, sb, db, sem):
            off = pl.multiple_of(a, 8)
            pltpu.async_copy(src_hbm.at[pl.ds(off, B)], sb, sem)
            pltpu.async_copy(dst_hbm.at[pl.ds(off, B)], db, sem)

        def idx_wait(sb, db, sem):
            pltpu.make_async_copy(src_hbm.at[pl.ds(0, B)], sb, sem).wait()
            pltpu.make_async_copy(dst_hbm.at[pl.ds(0, B)], db, sem).wait()

        for (c0, i0, o0, elo0), (c1, i1, o1, elo1) in rounds:
            cidx = jnp.where(core == 0, c0, c1)
            iidx = jnp.where(core == 0, i0, i1)
            obase = _sel8(core, o0 * NP, o1 * NP)
            elo = _sel8(core, elo0, elo1)
            pltpu.sync_copy(g_hbm.at[iidx].at[pl.ds(rlo, RPTA)],
                            acc.at[pl.ds(rlo, RPTA)])
            plsc.subcore_barrier()
            e0 = elo + sub * per

            def off(a):
                return e0 + jnp.minimum(a, nblk - 1) * B

            def gref(sb):
                return g_hbm.at[cidx].at[sb]

            def gat(sb, m, sem):
                pltpu.async_copy(gref(sb), m, sem)

            def gat_wait(sb, m, sem):
                pltpu.make_async_copy(gref(sb), m, sem).wait()

            def sca(m, db, sem):
                pltpu.async_copy(m, acc.at[db], sem, add=True)

            def sca_wait(m, db, sem):
                pltpu.make_async_copy(m, acc.at[db], sem).wait()

            idx_start(off(0), srcb0, dstb0, si0)
            idx_wait(srcb0, dstb0, si0)
            idx_start(off(1), srcb1, dstb1, si1)
            gat(srcb0, msg0, sg0)

            def pair(k, _):
                a = 2 * k
                gat_wait(srcb0, msg0, sg0)
                sca(msg0, dstb0, ss0)
                idx_wait(srcb1, dstb1, si1)
                gat(srcb1, msg1, sg1)
                sca_wait(msg0, dstb0, ss0)
                idx_start(off(a + 2), srcb0, dstb0, si0)
                gat_wait(srcb1, msg1, sg1)
                sca(msg1, dstb1, ss1)
                idx_wait(srcb0, dstb0, si0)
                gat(srcb0, msg0, sg0)
                sca_wait(msg1, dstb1, ss1)
                idx_start(off(a + 3), srcb1, dstb1, si1)
                return 0

            lax.fori_loop(0, pairs, pair, 0)
            gat_wait(srcb0, msg0, sg0)
            if odd:
                sca(msg0, dstb0, ss0)
                sca_wait(msg0, dstb0, ss0)
            idx_wait(srcb1, dstb1, si1)
            plsc.subcore_barrier()
            pltpu.sync_copy(acc.at[pl.ds(rlo, RPTA)],
                            out.at[pl.ds(obase + rlo, RPTA)])

    return pl.kernel(body, out_type=out_type, mesh=mesh,
                     scratch_types=scratch, compiler_params=_SC_PARAMS)


def _sc_hist(B=1000):
    D = 16
    mesh = plsc.VectorSubcoreMesh(core_axis_name="c", subcore_axis_name="s")
    out_type = jax.ShapeDtypeStruct((2 * NP, D), F32)
    scratch = [
        pltpu.VMEM((B,), jnp.int32),
        pltpu.VMEM((B, D), F32),
        pltpu.VMEM((ZR, D), F32),
        pltpu.VMEM_SHARED((RPT * NSUB, D), F32),
    ]
    per = (E // 2) // NSUB

    def body(dst_hbm, out, dstb, ones, zbuf, acc):
        core = lax.axis_index("c")
        sub = lax.axis_index("s")
        _zero_fill(zbuf, ZR, D)
        o = jnp.ones((16,), F32)

        def orow(i, _):
            ones[i, pl.ds(0, 16)] = o
            return 0

        lax.fori_loop(0, B, orow, 0)

        rlo = sub * RPT
        _zero_acc_slice(acc, zbuf, rlo, RPT)
        plsc.subcore_barrier()
        e0 = core * (E // 2) + sub * per

        def blk(i, _):
            off = pl.multiple_of(e0 + i * B, 8)
            pltpu.sync_copy(dst_hbm.at[pl.ds(off, B)], dstb)
            pltpu.sync_copy(ones, acc.at[dstb], add=True)
            return 0

        lax.fori_loop(0, per // B, blk, 0)
        plsc.subcore_barrier()
        obase = pl.multiple_of(core * NP, 8)
        pltpu.sync_copy(acc.at[pl.ds(rlo, RPT)],
                        out.at[pl.ds(obase + rlo, RPT)])

    return pl.kernel(body, out_type=out_type, mesh=mesh,
                     scratch_types=scratch, compiler_params=_SC_PARAMS)


def _rowspec(d, base_blocks=0):
    if base_blocks:
        return pl.BlockSpec((R, d), lambda i, b=base_blocks: (b + i, 0))
    return pl.BlockSpec((R, d), lambda i: (i, 0))


def _fullspec(r, c):
    return pl.BlockSpec((r, c), lambda i: (0, 0))


def _tc1(hist, x):
    def body(h0_, h1_, x_, dinv, g0):
        deg = h0_[:, 0:1] + h1_[:, 0:1] + 1.0
        di = lax.rsqrt(deg)
        dinv[...] = di
        g = x_[...] * di
        g0[...] = jnp.stack(
            [g[:, 0:32],
             jnp.concatenate([g[:, 32:58], jnp.zeros((R, 6), F32)], axis=1)])

    return pl.pallas_call(
        body,
        grid=(G,),
        in_specs=[_rowspec(16), _rowspec(16, NPB), _rowspec(58)],
        out_specs=[_rowspec(1), pl.BlockSpec((2, R, 32), lambda i: (0, i, 0))],
        out_shape=[jax.ShapeDtypeStruct((N, 1), F32),
                   jax.ShapeDtypeStruct((2, NA, 32), F32)],
    )(hist, hist, x)


def _tc2(s0, dinv, W1p, b1p, W2p):
    def body(s0a, s0b, di_, W1_, b1_, W2_, g1):
        di = di_[...]
        a = jnp.concatenate([s0a[...], s0b[...]], axis=1) * di
        h = jnp.maximum(
            jnp.dot(a, W1_[...], preferred_element_type=F32) + b1_[...], 0.0)
        p = jnp.dot(h, W2_[...], preferred_element_type=F32) * di
        g1[...] = jnp.stack([p[:, 32 * c:32 * c + 32] for c in range(4)])

    return pl.pallas_call(
        body,
        grid=(G,),
        in_specs=[_rowspec(32), _rowspec(32, NPB),
                  _rowspec(1), _fullspec(64, 384), _fullspec(1, 384),
                  _fullspec(384, 128)],
        out_specs=[pl.BlockSpec((4, R, 32), lambda i: (0, i, 0))],
        out_shape=[jax.ShapeDtypeStruct((4, NA, 32), F32)],
    )(s0, s0, dinv, W1p, b1p, W2p)


def _tc3(s1a, s1b, dinv, b2p, W3p):
    def body(sa, sb, sc, sd, di_, b2_, W3_, out):
        di = di_[...]
        a = jnp.concatenate(
            [sa[...], sb[...], sc[...], sd[...]], axis=1) * di
        h = jnp.maximum(a + b2_[...], 0.0)
        g2 = jnp.dot(h, W3_[...], preferred_element_type=F32) * di
        out[...] = jnp.stack([g2, jnp.zeros((R, 16), F32)])

    return pl.pallas_call(
        body,
        grid=(G,),
        in_specs=[_rowspec(32), _rowspec(32, NPB), _rowspec(32),
                  _rowspec(32, NPB),
                  _rowspec(1), _fullspec(1, 128), _fullspec(128, 16)],
        out_specs=[pl.BlockSpec((2, R, 16), lambda i: (0, i, 0))],
        out_shape=[jax.ShapeDtypeStruct((2, NA, 16), F32)],
    )(s1a, s1a, s1b, s1b, dinv, b2p, W3p)


def _tc4(s2, dinv, b3p):
    def body(sa, sb, di_, b3_, out):
        t = (sa[...] + sb[...]) * di_[...]
        out[...] = t[:, 0:1] + b3_[...]

    return pl.pallas_call(
        body,
        grid=(G,),
        in_specs=[_rowspec(16), _rowspec(16, NPB), _rowspec(1),
                  _fullspec(1, 1)],
        out_specs=[_rowspec(1)],
        out_shape=[jax.ShapeDtypeStruct((N, 1), F32)],
    )(s2, s2, dinv, b3p)


_hist = _sc_hist()
_agg1 = _sc_agg(n_g=2, n_out=2, D=32, B=400,
                rounds=[((0, 0, 0, 0), (1, 1, 1, 0))], ecount=E)
_agg2a = _sc_agg(n_g=4, n_out=2, D=32, B=400,
                 rounds=[((0, 0, 0, 0), (1, 1, 1, 0))], ecount=E)
_agg2b = _sc_agg(n_g=4, n_out=2, D=32, B=400,
                 rounds=[((2, 2, 0, 0), (3, 3, 1, 0))], ecount=E)
_agg3 = _sc_agg(n_g=2, n_out=2, D=16, B=1000,
                rounds=[((0, 0, 0, 0), (0, 1, 1, E // 2))], ecount=E // 2)


def kernel(x, edge_index, W1, b1, W2, b2, W3, b3):
    src = edge_index[0]
    dst = edge_index[1]
    W1p = jnp.zeros((64, 384), F32).at[:58, :300].set(W1)
    b1p = jnp.zeros((1, 384), F32).at[0, :300].set(b1)
    W2p = jnp.zeros((384, 128), F32).at[:300, :100].set(W2)
    b2p = jnp.zeros((1, 128), F32).at[0, :100].set(b2)
    W3p = jnp.zeros((128, 16), F32).at[:100, 0:1].set(W3)
    b3p = b3.reshape(1, 1)

    hist = _hist(dst)
    dinv, g0 = _tc1(hist, x)
    s0 = _agg1(g0, src, dst)
    (g1,) = _tc2(s0, dinv, W1p, b1p, W2p)
    s1a = _agg2a(g1, src, dst)
    s1b = _agg2b(g1, src, dst)
    (g2z,) = _tc3(s1a, s1b, dinv, b2p, W3p)
    s2 = _agg3(g2z, src, dst)
    (out,) = _tc4(s2, dinv, b3p)
    return out
